# baseline (speedup 1.0000x reference)
import functools

import jax
import jax.numpy as jnp
import numpy as np
from jax.experimental import pallas as pl
from jax.experimental.pallas import tpu as pltpu

BM = 512


def _keys_body(fo_ref, fr_ref, wa1_ref, ba1_ref, wa2_ref, ba2_ref,
               k1t_ref, k2t_ref):
    rr = jnp.maximum(fr_ref[...], 0.0)
    k1t_ref[...] = (jax.lax.dot_general(
        wa1_ref[...], rr, (((1,), (1,)), ((), ())),
        preferred_element_type=jnp.float32)
        + ba1_ref[...]).astype(jnp.bfloat16)
    ro = jnp.maximum(fo_ref[...], 0.0)
    k2t_ref[...] = (jax.lax.dot_general(
        wa2_ref[...], ro, (((1,), (1,)), ((), ())),
        preferred_element_type=jnp.float32)
        + ba2_ref[...]).astype(jnp.bfloat16)


def _attn_body(qscale,
               fo_ref, fr_ref, fo1_ref, fr1_ref, k1t_ref, k2t_ref,
               mo_ref, mr_ref,
               wq1_ref, bq1_ref, wq2_ref, bq2_ref,
               wt_r2o_ref, bt_r2o_ref, wt_o2r_ref, bt_o2r_ref,
               out_obj_ref, out_reg_ref):
    i = pl.program_id(0)
    fo_blk = fo_ref[pl.ds(i * BM, BM), :]
    fr_blk = fr_ref[pl.ds(i * BM, BM), :]

    q1 = jax.lax.dot_general(
        jnp.maximum(fo_blk, 0.0), wq1_ref[...], (((1,), (1,)), ((), ())),
        preferred_element_type=jnp.float32) + bq1_ref[...]
    q1 = (q1 * qscale).astype(jnp.bfloat16)
    s1 = jnp.dot(q1, k1t_ref[...], preferred_element_type=jnp.float32)
    ef1 = jnp.exp2(s1) * mo_ref[...].astype(jnp.float32)
    den1 = jnp.sum(ef1, axis=1, keepdims=True)
    e1 = ef1.astype(jnp.bfloat16)
    msg1 = jnp.dot(e1, fr1_ref[...], preferred_element_type=jnp.float32)
    msg1 = msg1 * (1.0 / den1)
    out_obj_ref[...] = fo_blk + jax.lax.dot_general(
        jnp.maximum(msg1, 0.0), wt_r2o_ref[...], (((1,), (1,)), ((), ())),
        preferred_element_type=jnp.float32) + bt_r2o_ref[...]

    q2 = jax.lax.dot_general(
        jnp.maximum(fr_blk, 0.0), wq2_ref[...], (((1,), (1,)), ((), ())),
        preferred_element_type=jnp.float32) + bq2_ref[...]
    q2 = (q2 * qscale).astype(jnp.bfloat16)
    s2 = jnp.dot(q2, k2t_ref[...], preferred_element_type=jnp.float32)
    ef2 = jnp.exp2(s2) * mr_ref[...].astype(jnp.float32)
    den2 = jnp.sum(ef2, axis=1, keepdims=True)
    e2 = ef2.astype(jnp.bfloat16)
    msg2 = jnp.dot(e2, fo1_ref[...], preferred_element_type=jnp.float32)
    msg2 = msg2 * (1.0 / den2)
    out_reg_ref[...] = fr_blk + jax.lax.dot_general(
        jnp.maximum(msg2, 0.0), wt_o2r_ref[...], (((1,), (1,)), ((), ())),
        preferred_element_type=jnp.float32) + bt_o2r_ref[...]


def kernel(feature_obj, feature_region, mat_object, mat_region,
           Wt_o2r, bt_o2r, Wt_r2o, bt_r2o,
           Wa_r2o_obj, ba_r2o_obj, Wa_r2o_reg, ba_r2o_reg,
           Wa_o2r_reg, ba_o2r_reg, Wa_o2r_obj, ba_o2r_obj):
    no, dho = feature_obj.shape
    nr, dhr, h, w = feature_region.shape
    dmm = Wa_r2o_obj.shape[0]
    fr2d = feature_region.reshape(nr, dhr)
    fo1 = feature_obj.astype(jnp.bfloat16)
    fr1 = fr2d.astype(jnp.bfloat16)
    qscale = np.float32((1.0 / np.sqrt(dmm + 1e-10)) * np.log2(np.e))

    full = lambda shp: pl.BlockSpec(shp, lambda i: (0, 0))

    k1t, k2t = pl.pallas_call(
        _keys_body,
        grid=(),
        in_specs=[pl.BlockSpec(feature_obj.shape, lambda: (0, 0)),
                  pl.BlockSpec(fr2d.shape, lambda: (0, 0)),
                  pl.BlockSpec(Wa_r2o_reg.shape, lambda: (0, 0)),
                  pl.BlockSpec((dmm, 1), lambda: (0, 0)),
                  pl.BlockSpec(Wa_o2r_obj.shape, lambda: (0, 0)),
                  pl.BlockSpec((dmm, 1), lambda: (0, 0))],
        out_specs=[pl.BlockSpec((dmm, nr), lambda: (0, 0)),
                   pl.BlockSpec((dmm, no), lambda: (0, 0))],
        out_shape=[jax.ShapeDtypeStruct((dmm, nr), jnp.bfloat16),
                   jax.ShapeDtypeStruct((dmm, no), jnp.bfloat16)],
    )(feature_obj, fr2d,
      Wa_r2o_reg, ba_r2o_reg.reshape(dmm, 1),
      Wa_o2r_obj, ba_o2r_obj.reshape(dmm, 1))

    grid = (no // BM,)
    out_obj, out_reg2d = pl.pallas_call(
        functools.partial(_attn_body, qscale),
        grid=grid,
        in_specs=[full(feature_obj.shape),
                  full(fr2d.shape),
                  full(fo1.shape),
                  full(fr1.shape),
                  full((dmm, nr)),
                  full((dmm, no)),
                  pl.BlockSpec((BM, nr), lambda i: (i, 0)),
                  pl.BlockSpec((BM, no), lambda i: (i, 0)),
                  full(Wa_r2o_obj.shape),
                  full((1, dmm)),
                  full(Wa_o2r_reg.shape),
                  full((1, dmm)),
                  full(Wt_r2o.shape),
                  full((1, dho)),
                  full(Wt_o2r.shape),
                  full((1, dhr))],
        out_specs=[pl.BlockSpec((BM, dho), lambda i: (i, 0)),
                   pl.BlockSpec((BM, dhr), lambda i: (i, 0))],
        out_shape=[jax.ShapeDtypeStruct((no, dho), jnp.float32),
                   jax.ShapeDtypeStruct((nr, dhr), jnp.float32)],
        compiler_params=pltpu.CompilerParams(
            dimension_semantics=("parallel",)),
    )(feature_obj, fr2d, fo1, fr1, k1t, k2t, mat_object, mat_region,
      Wa_r2o_obj, ba_r2o_obj.reshape(1, dmm),
      Wa_o2r_reg, ba_o2r_reg.reshape(1, dmm),
      Wt_r2o, bt_r2o.reshape(1, dho),
      Wt_o2r, bt_o2r.reshape(1, dhr))

    return (out_obj, out_reg2d.reshape(nr, dhr, h, w))


# speedup vs baseline: 1.0884x; 1.0884x over previous
import functools

import jax
import jax.numpy as jnp
import numpy as np
from jax.experimental import pallas as pl
from jax.experimental.pallas import tpu as pltpu

BM = 128


def _keys_body(fo_ref, fr_ref, wa1_ref, ba1_ref, wa2_ref, ba2_ref,
               k1t_ref, k2t_ref):
    rr = jnp.maximum(fr_ref[...], 0.0)
    k1t_ref[...] = (jax.lax.dot_general(
        wa1_ref[...], rr, (((1,), (1,)), ((), ())),
        preferred_element_type=jnp.float32)
        + ba1_ref[...]).astype(jnp.bfloat16)
    ro = jnp.maximum(fo_ref[...], 0.0)
    k2t_ref[...] = (jax.lax.dot_general(
        wa2_ref[...], ro, (((1,), (1,)), ((), ())),
        preferred_element_type=jnp.float32)
        + ba2_ref[...]).astype(jnp.bfloat16)


def _attn_body(qscale,
               fo_ref, fr_ref, fo1_ref, fr1_ref, k1t_ref, k2t_ref,
               mo_ref, mr_ref,
               wq1_ref, bq1_ref, wq2_ref, bq2_ref,
               wt_r2o_ref, bt_r2o_ref, wt_o2r_ref, bt_o2r_ref,
               out_obj_ref, out_reg_ref):
    i = pl.program_id(0)
    fo_blk = fo_ref[pl.ds(i * BM, BM), :]
    fr_blk = fr_ref[pl.ds(i * BM, BM), :]

    q1 = jax.lax.dot_general(
        jnp.maximum(fo_blk, 0.0), wq1_ref[...], (((1,), (1,)), ((), ())),
        preferred_element_type=jnp.float32) + bq1_ref[...]
    q1 = (q1 * qscale).astype(jnp.bfloat16)
    s1 = jnp.dot(q1, k1t_ref[...], preferred_element_type=jnp.float32)
    ef1 = jnp.exp2(s1) * mo_ref[...].astype(jnp.float32)
    den1 = jnp.sum(ef1, axis=1, keepdims=True)
    e1 = ef1.astype(jnp.bfloat16)
    msg1 = jnp.dot(e1, fr1_ref[...], preferred_element_type=jnp.float32)
    msg1 = msg1 * (1.0 / den1)
    out_obj_ref[...] = fo_blk + jax.lax.dot_general(
        jnp.maximum(msg1, 0.0), wt_r2o_ref[...], (((1,), (1,)), ((), ())),
        preferred_element_type=jnp.float32) + bt_r2o_ref[...]

    q2 = jax.lax.dot_general(
        jnp.maximum(fr_blk, 0.0), wq2_ref[...], (((1,), (1,)), ((), ())),
        preferred_element_type=jnp.float32) + bq2_ref[...]
    q2 = (q2 * qscale).astype(jnp.bfloat16)
    s2 = jnp.dot(q2, k2t_ref[...], preferred_element_type=jnp.float32)
    ef2 = jnp.exp2(s2) * mr_ref[...].astype(jnp.float32)
    den2 = jnp.sum(ef2, axis=1, keepdims=True)
    e2 = ef2.astype(jnp.bfloat16)
    msg2 = jnp.dot(e2, fo1_ref[...], preferred_element_type=jnp.float32)
    msg2 = msg2 * (1.0 / den2)
    out_reg_ref[...] = fr_blk + jax.lax.dot_general(
        jnp.maximum(msg2, 0.0), wt_o2r_ref[...], (((1,), (1,)), ((), ())),
        preferred_element_type=jnp.float32) + bt_o2r_ref[...]


def kernel(feature_obj, feature_region, mat_object, mat_region,
           Wt_o2r, bt_o2r, Wt_r2o, bt_r2o,
           Wa_r2o_obj, ba_r2o_obj, Wa_r2o_reg, ba_r2o_reg,
           Wa_o2r_reg, ba_o2r_reg, Wa_o2r_obj, ba_o2r_obj):
    no, dho = feature_obj.shape
    nr, dhr, h, w = feature_region.shape
    dmm = Wa_r2o_obj.shape[0]
    fr2d = feature_region.reshape(nr, dhr)
    fo1 = feature_obj.astype(jnp.bfloat16)
    fr1 = fr2d.astype(jnp.bfloat16)
    qscale = np.float32((1.0 / np.sqrt(dmm + 1e-10)) * np.log2(np.e))

    full = lambda shp: pl.BlockSpec(shp, lambda i: (0, 0))

    k1t, k2t = pl.pallas_call(
        _keys_body,
        grid=(),
        in_specs=[pl.BlockSpec(feature_obj.shape, lambda: (0, 0)),
                  pl.BlockSpec(fr2d.shape, lambda: (0, 0)),
                  pl.BlockSpec(Wa_r2o_reg.shape, lambda: (0, 0)),
                  pl.BlockSpec((dmm, 1), lambda: (0, 0)),
                  pl.BlockSpec(Wa_o2r_obj.shape, lambda: (0, 0)),
                  pl.BlockSpec((dmm, 1), lambda: (0, 0))],
        out_specs=[pl.BlockSpec((dmm, nr), lambda: (0, 0)),
                   pl.BlockSpec((dmm, no), lambda: (0, 0))],
        out_shape=[jax.ShapeDtypeStruct((dmm, nr), jnp.bfloat16),
                   jax.ShapeDtypeStruct((dmm, no), jnp.bfloat16)],
    )(feature_obj, fr2d,
      Wa_r2o_reg, ba_r2o_reg.reshape(dmm, 1),
      Wa_o2r_obj, ba_o2r_obj.reshape(dmm, 1))

    grid = (no // BM,)
    out_obj, out_reg2d = pl.pallas_call(
        functools.partial(_attn_body, qscale),
        grid=grid,
        in_specs=[full(feature_obj.shape),
                  full(fr2d.shape),
                  full(fo1.shape),
                  full(fr1.shape),
                  full((dmm, nr)),
                  full((dmm, no)),
                  pl.BlockSpec((BM, nr), lambda i: (i, 0)),
                  pl.BlockSpec((BM, no), lambda i: (i, 0)),
                  full(Wa_r2o_obj.shape),
                  full((1, dmm)),
                  full(Wa_o2r_reg.shape),
                  full((1, dmm)),
                  full(Wt_r2o.shape),
                  full((1, dho)),
                  full(Wt_o2r.shape),
                  full((1, dhr))],
        out_specs=[pl.BlockSpec((BM, dho), lambda i: (i, 0)),
                   pl.BlockSpec((BM, dhr), lambda i: (i, 0))],
        out_shape=[jax.ShapeDtypeStruct((no, dho), jnp.float32),
                   jax.ShapeDtypeStruct((nr, dhr), jnp.float32)],
        compiler_params=pltpu.CompilerParams(
            dimension_semantics=("parallel",)),
    )(feature_obj, fr2d, fo1, fr1, k1t, k2t, mat_object, mat_region,
      Wa_r2o_obj, ba_r2o_obj.reshape(1, dmm),
      Wa_o2r_reg, ba_o2r_reg.reshape(1, dmm),
      Wt_r2o, bt_r2o.reshape(1, dho),
      Wt_o2r, bt_o2r.reshape(1, dhr))

    return (out_obj, out_reg2d.reshape(nr, dhr, h, w))


# speedup vs baseline: 1.2286x; 1.1288x over previous
import functools

import jax
import jax.numpy as jnp
import numpy as np
from jax.experimental import pallas as pl
from jax.experimental.pallas import tpu as pltpu

BM = 256


def _keys_body(fo_ref, fr_ref, wa1_ref, ba1_ref, wa2_ref, ba2_ref,
               k1t_ref, k2t_ref):
    rr = jnp.maximum(fr_ref[...], 0.0)
    k1t_ref[...] = (jax.lax.dot_general(
        wa1_ref[...], rr, (((1,), (1,)), ((), ())),
        preferred_element_type=jnp.float32)
        + ba1_ref[...]).astype(jnp.bfloat16)
    ro = jnp.maximum(fo_ref[...], 0.0)
    k2t_ref[...] = (jax.lax.dot_general(
        wa2_ref[...], ro, (((1,), (1,)), ((), ())),
        preferred_element_type=jnp.float32)
        + ba2_ref[...]).astype(jnp.bfloat16)


def _attn_body(qscale,
               fo_ref, fr_ref, fo1_ref, fr1_ref, k1t_ref, k2t_ref,
               mo_ref, mr_ref,
               wq1_ref, bq1_ref, wq2_ref, bq2_ref,
               wt_r2o_ref, bt_r2o_ref, wt_o2r_ref, bt_o2r_ref,
               out_obj_ref, out_reg_ref):
    i = pl.program_id(0)
    fo_blk = fo_ref[pl.ds(i * BM, BM), :]
    fr_blk = fr_ref[pl.ds(i * BM, BM), :]

    q1 = jax.lax.dot_general(
        jnp.maximum(fo_blk, 0.0), wq1_ref[...], (((1,), (1,)), ((), ())),
        preferred_element_type=jnp.float32) + bq1_ref[...]
    q1 = (q1 * qscale).astype(jnp.bfloat16)
    s1 = jnp.dot(q1, k1t_ref[...], preferred_element_type=jnp.float32)
    ef1 = jnp.exp2(s1) * mo_ref[...].astype(jnp.float32)
    den1 = jnp.sum(ef1, axis=1, keepdims=True)
    e1 = ef1.astype(jnp.bfloat16)
    msg1 = jnp.dot(e1, fr1_ref[...], preferred_element_type=jnp.float32)
    msg1 = msg1 * (1.0 / den1)
    out_obj_ref[...] = fo_blk + jax.lax.dot_general(
        jnp.maximum(msg1, 0.0), wt_r2o_ref[...], (((1,), (1,)), ((), ())),
        preferred_element_type=jnp.float32) + bt_r2o_ref[...]

    q2 = jax.lax.dot_general(
        jnp.maximum(fr_blk, 0.0), wq2_ref[...], (((1,), (1,)), ((), ())),
        preferred_element_type=jnp.float32) + bq2_ref[...]
    q2 = (q2 * qscale).astype(jnp.bfloat16)
    s2 = jnp.dot(q2, k2t_ref[...], preferred_element_type=jnp.float32)
    ef2 = jnp.exp2(s2) * mr_ref[...].astype(jnp.float32)
    den2 = jnp.sum(ef2, axis=1, keepdims=True)
    e2 = ef2.astype(jnp.bfloat16)
    msg2 = jnp.dot(e2, fo1_ref[...], preferred_element_type=jnp.float32)
    msg2 = msg2 * (1.0 / den2)
    out_reg_ref[...] = fr_blk + jax.lax.dot_general(
        jnp.maximum(msg2, 0.0), wt_o2r_ref[...], (((1,), (1,)), ((), ())),
        preferred_element_type=jnp.float32) + bt_o2r_ref[...]


def kernel(feature_obj, feature_region, mat_object, mat_region,
           Wt_o2r, bt_o2r, Wt_r2o, bt_r2o,
           Wa_r2o_obj, ba_r2o_obj, Wa_r2o_reg, ba_r2o_reg,
           Wa_o2r_reg, ba_o2r_reg, Wa_o2r_obj, ba_o2r_obj):
    no, dho = feature_obj.shape
    nr, dhr, h, w = feature_region.shape
    dmm = Wa_r2o_obj.shape[0]
    fr2d = feature_region.reshape(nr, dhr)
    fo1 = feature_obj.astype(jnp.bfloat16)
    fr1 = fr2d.astype(jnp.bfloat16)
    qscale = np.float32((1.0 / np.sqrt(dmm + 1e-10)) * np.log2(np.e))

    full = lambda shp: pl.BlockSpec(shp, lambda i: (0, 0))

    k1t, k2t = pl.pallas_call(
        _keys_body,
        grid=(),
        in_specs=[pl.BlockSpec(feature_obj.shape, lambda: (0, 0)),
                  pl.BlockSpec(fr2d.shape, lambda: (0, 0)),
                  pl.BlockSpec(Wa_r2o_reg.shape, lambda: (0, 0)),
                  pl.BlockSpec((dmm, 1), lambda: (0, 0)),
                  pl.BlockSpec(Wa_o2r_obj.shape, lambda: (0, 0)),
                  pl.BlockSpec((dmm, 1), lambda: (0, 0))],
        out_specs=[pl.BlockSpec((dmm, nr), lambda: (0, 0)),
                   pl.BlockSpec((dmm, no), lambda: (0, 0))],
        out_shape=[jax.ShapeDtypeStruct((dmm, nr), jnp.bfloat16),
                   jax.ShapeDtypeStruct((dmm, no), jnp.bfloat16)],
    )(feature_obj, fr2d,
      Wa_r2o_reg, ba_r2o_reg.reshape(dmm, 1),
      Wa_o2r_obj, ba_o2r_obj.reshape(dmm, 1))

    grid = (no // BM,)
    out_obj, out_reg2d = pl.pallas_call(
        functools.partial(_attn_body, qscale),
        grid=grid,
        in_specs=[full(feature_obj.shape),
                  full(fr2d.shape),
                  full(fo1.shape),
                  full(fr1.shape),
                  full((dmm, nr)),
                  full((dmm, no)),
                  pl.BlockSpec((BM, nr), lambda i: (i, 0)),
                  pl.BlockSpec((BM, no), lambda i: (i, 0)),
                  full(Wa_r2o_obj.shape),
                  full((1, dmm)),
                  full(Wa_o2r_reg.shape),
                  full((1, dmm)),
                  full(Wt_r2o.shape),
                  full((1, dho)),
                  full(Wt_o2r.shape),
                  full((1, dhr))],
        out_specs=[pl.BlockSpec((BM, dho), lambda i: (i, 0)),
                   pl.BlockSpec((BM, dhr), lambda i: (i, 0))],
        out_shape=[jax.ShapeDtypeStruct((no, dho), jnp.float32),
                   jax.ShapeDtypeStruct((nr, dhr), jnp.float32)],
        compiler_params=pltpu.CompilerParams(
            dimension_semantics=("parallel",)),
    )(feature_obj, fr2d, fo1, fr1, k1t, k2t, mat_object, mat_region,
      Wa_r2o_obj, ba_r2o_obj.reshape(1, dmm),
      Wa_o2r_reg, ba_o2r_reg.reshape(1, dmm),
      Wt_r2o, bt_r2o.reshape(1, dho),
      Wt_o2r, bt_o2r.reshape(1, dhr))

    return (out_obj, out_reg2d.reshape(nr, dhr, h, w))


# speedup vs baseline: 1.3436x; 1.0936x over previous
import functools

import jax
import jax.numpy as jnp
import numpy as np
from jax.experimental import pallas as pl
from jax.experimental.pallas import tpu as pltpu

BM = 256


def _attn_body(qscale,
               fo_ref, fr_ref,
               mo_ref, mr_ref,
               wq1_ref, bq1_ref, wq2_ref, bq2_ref,
               wk1_ref, bk1_ref, wk2_ref, bk2_ref,
               wt_r2o_ref, bt_r2o_ref, wt_o2r_ref, bt_o2r_ref,
               out_obj_ref, out_reg_ref,
               fo1_s, fr1_s, k1t_s, k2t_s):
    i = pl.program_id(0)

    @pl.when(i == 0)
    def _build_shared():
        rr = jnp.maximum(fr_ref[...], 0.0)
        k1t_s[...] = (jax.lax.dot_general(
            wk1_ref[...], rr, (((1,), (1,)), ((), ())),
            preferred_element_type=jnp.float32)
            + bk1_ref[...]).astype(jnp.bfloat16)
        ro = jnp.maximum(fo_ref[...], 0.0)
        k2t_s[...] = (jax.lax.dot_general(
            wk2_ref[...], ro, (((1,), (1,)), ((), ())),
            preferred_element_type=jnp.float32)
            + bk2_ref[...]).astype(jnp.bfloat16)
        fo1_s[...] = fo_ref[...].astype(jnp.bfloat16)
        fr1_s[...] = fr_ref[...].astype(jnp.bfloat16)

    fo_blk = fo_ref[pl.ds(i * BM, BM), :]
    fr_blk = fr_ref[pl.ds(i * BM, BM), :]

    q1 = jax.lax.dot_general(
        jnp.maximum(fo_blk, 0.0), wq1_ref[...], (((1,), (1,)), ((), ())),
        preferred_element_type=jnp.float32) + bq1_ref[...]
    q1 = (q1 * qscale).astype(jnp.bfloat16)
    s1 = jnp.dot(q1, k1t_s[...], preferred_element_type=jnp.float32)
    ef1 = jnp.exp2(s1) * mo_ref[...].astype(jnp.float32)
    den1 = jnp.sum(ef1, axis=1, keepdims=True)
    e1 = ef1.astype(jnp.bfloat16)
    msg1 = jnp.dot(e1, fr1_s[...], preferred_element_type=jnp.float32)
    msg1 = msg1 * (1.0 / den1)
    out_obj_ref[...] = fo_blk + jax.lax.dot_general(
        jnp.maximum(msg1, 0.0), wt_r2o_ref[...], (((1,), (1,)), ((), ())),
        preferred_element_type=jnp.float32) + bt_r2o_ref[...]

    q2 = jax.lax.dot_general(
        jnp.maximum(fr_blk, 0.0), wq2_ref[...], (((1,), (1,)), ((), ())),
        preferred_element_type=jnp.float32) + bq2_ref[...]
    q2 = (q2 * qscale).astype(jnp.bfloat16)
    s2 = jnp.dot(q2, k2t_s[...], preferred_element_type=jnp.float32)
    ef2 = jnp.exp2(s2) * mr_ref[...].astype(jnp.float32)
    den2 = jnp.sum(ef2, axis=1, keepdims=True)
    e2 = ef2.astype(jnp.bfloat16)
    msg2 = jnp.dot(e2, fo1_s[...], preferred_element_type=jnp.float32)
    msg2 = msg2 * (1.0 / den2)
    out_reg_ref[...] = fr_blk + jax.lax.dot_general(
        jnp.maximum(msg2, 0.0), wt_o2r_ref[...], (((1,), (1,)), ((), ())),
        preferred_element_type=jnp.float32) + bt_o2r_ref[...]


def kernel(feature_obj, feature_region, mat_object, mat_region,
           Wt_o2r, bt_o2r, Wt_r2o, bt_r2o,
           Wa_r2o_obj, ba_r2o_obj, Wa_r2o_reg, ba_r2o_reg,
           Wa_o2r_reg, ba_o2r_reg, Wa_o2r_obj, ba_o2r_obj):
    no, dho = feature_obj.shape
    nr, dhr, h, w = feature_region.shape
    dmm = Wa_r2o_obj.shape[0]
    fr2d = feature_region.reshape(nr, dhr)
    qscale = np.float32((1.0 / np.sqrt(dmm + 1e-10)) * np.log2(np.e))

    full = lambda shp: pl.BlockSpec(shp, lambda i: (0, 0))

    grid = (no // BM,)
    out_obj, out_reg2d = pl.pallas_call(
        functools.partial(_attn_body, qscale),
        grid=grid,
        in_specs=[full(feature_obj.shape),
                  full(fr2d.shape),
                  pl.BlockSpec((BM, nr), lambda i: (i, 0)),
                  pl.BlockSpec((BM, no), lambda i: (i, 0)),
                  full(Wa_r2o_obj.shape),
                  full((1, dmm)),
                  full(Wa_o2r_reg.shape),
                  full((1, dmm)),
                  full(Wa_r2o_reg.shape),
                  full((dmm, 1)),
                  full(Wa_o2r_obj.shape),
                  full((dmm, 1)),
                  full(Wt_r2o.shape),
                  full((1, dho)),
                  full(Wt_o2r.shape),
                  full((1, dhr))],
        out_specs=[pl.BlockSpec((BM, dho), lambda i: (i, 0)),
                   pl.BlockSpec((BM, dhr), lambda i: (i, 0))],
        out_shape=[jax.ShapeDtypeStruct((no, dho), jnp.float32),
                   jax.ShapeDtypeStruct((nr, dhr), jnp.float32)],
        scratch_shapes=[pltpu.VMEM((no, dho), jnp.bfloat16),
                        pltpu.VMEM((nr, dhr), jnp.bfloat16),
                        pltpu.VMEM((dmm, nr), jnp.bfloat16),
                        pltpu.VMEM((dmm, no), jnp.bfloat16)],
        compiler_params=pltpu.CompilerParams(
            dimension_semantics=("arbitrary",)),
    )(feature_obj, fr2d, mat_object, mat_region,
      Wa_r2o_obj, ba_r2o_obj.reshape(1, dmm),
      Wa_o2r_reg, ba_o2r_reg.reshape(1, dmm),
      Wa_r2o_reg, ba_r2o_reg.reshape(dmm, 1),
      Wa_o2r_obj, ba_o2r_obj.reshape(dmm, 1),
      Wt_r2o, bt_r2o.reshape(1, dho),
      Wt_o2r, bt_o2r.reshape(1, dhr))

    return (out_obj, out_reg2d.reshape(nr, dhr, h, w))
